# baseline (speedup 1.0000x reference)
import functools

import jax
import jax.numpy as jnp
from jax.experimental import pallas as pl
from jax.experimental.pallas import tpu as pltpu

NUM_EXPERTS = 64
TOP_K = 2


def _router_block(x_ref, wt_ref, w_out_ref, i_out_ref, s_out_ref):
    logits = jnp.dot(x_ref[...], wt_ref[...], preferred_element_type=jnp.float32)
    m = jnp.max(logits, axis=-1, keepdims=True)
    e = jnp.exp(logits - m)
    s = jnp.sum(e, axis=-1, keepdims=True)
    scores = e / s
    s_out_ref[...] = scores

    i1 = jnp.argmax(scores, axis=-1)
    m1 = jnp.max(scores, axis=-1)
    cols = jax.lax.broadcasted_iota(jnp.int32, scores.shape, 1)
    masked = jnp.where(cols == i1[:, None], -jnp.inf, scores)
    i2 = jnp.argmax(masked, axis=-1)
    m2 = jnp.max(masked, axis=-1)
    w_out_ref[...] = jnp.stack([m1, m2], axis=-1)
    i_out_ref[...] = jnp.stack([i1, i2], axis=-1).astype(jnp.int32)


@functools.partial(jax.jit, static_argnames=())
def kernel(x, W):
    n_tokens, d_model = x.shape
    wt = W.T
    bt = 512
    grid = (n_tokens // bt,)
    weights, indices, scores = pl.pallas_call(
        _router_block,
        grid=grid,
        in_specs=[
            pl.BlockSpec((bt, d_model), lambda i: (i, 0)),
            pl.BlockSpec((d_model, NUM_EXPERTS), lambda i: (0, 0)),
        ],
        out_specs=[
            pl.BlockSpec((bt, TOP_K), lambda i: (i, 0)),
            pl.BlockSpec((bt, TOP_K), lambda i: (i, 0)),
            pl.BlockSpec((bt, NUM_EXPERTS), lambda i: (i, 0)),
        ],
        out_shape=[
            jax.ShapeDtypeStruct((n_tokens, TOP_K), jnp.float32),
            jax.ShapeDtypeStruct((n_tokens, TOP_K), jnp.int32),
            jax.ShapeDtypeStruct((n_tokens, NUM_EXPERTS), jnp.float32),
        ],
        compiler_params=pltpu.CompilerParams(
            dimension_semantics=("parallel",),
        ),
    )(x, wt)
    return weights, indices, scores


# speedup vs baseline: 1.1000x; 1.1000x over previous
import functools

import jax
import jax.numpy as jnp
from jax.experimental import pallas as pl
from jax.experimental.pallas import tpu as pltpu

NUM_EXPERTS = 64
TOP_K = 2


def _router_block(x_ref, wt_ref, w_out_ref, i_out_ref, s_out_ref):
    logits = jnp.dot(x_ref[...], wt_ref[...], preferred_element_type=jnp.float32)
    m = jnp.max(logits, axis=-1, keepdims=True)
    e = jnp.exp(logits - m)
    s = jnp.sum(e, axis=-1, keepdims=True)
    scores = e / s
    s_out_ref[...] = scores

    i1 = jnp.argmax(scores, axis=-1)
    m1 = jnp.max(scores, axis=-1)
    cols = jax.lax.broadcasted_iota(jnp.int32, scores.shape, 1)
    masked = jnp.where(cols == i1[:, None], -jnp.inf, scores)
    i2 = jnp.argmax(masked, axis=-1)
    m2 = jnp.max(masked, axis=-1)
    w_out_ref[...] = jnp.stack([m1, m2], axis=-1)
    i_out_ref[...] = jnp.stack([i1, i2], axis=-1).astype(jnp.int32)


@functools.partial(jax.jit, static_argnames=())
def kernel(x, W):
    n_tokens, d_model = x.shape
    wt = W.T
    bt = 2048
    grid = (n_tokens // bt,)
    weights, indices, scores = pl.pallas_call(
        _router_block,
        grid=grid,
        in_specs=[
            pl.BlockSpec((bt, d_model), lambda i: (i, 0)),
            pl.BlockSpec((d_model, NUM_EXPERTS), lambda i: (0, 0)),
        ],
        out_specs=[
            pl.BlockSpec((bt, TOP_K), lambda i: (i, 0)),
            pl.BlockSpec((bt, TOP_K), lambda i: (i, 0)),
            pl.BlockSpec((bt, NUM_EXPERTS), lambda i: (i, 0)),
        ],
        out_shape=[
            jax.ShapeDtypeStruct((n_tokens, TOP_K), jnp.float32),
            jax.ShapeDtypeStruct((n_tokens, TOP_K), jnp.int32),
            jax.ShapeDtypeStruct((n_tokens, NUM_EXPERTS), jnp.float32),
        ],
        compiler_params=pltpu.CompilerParams(
            dimension_semantics=("parallel",),
        ),
    )(x, wt)
    return weights, indices, scores
